# baseline (speedup 1.0000x reference)
import functools

import jax
import jax.numpy as jnp
from jax import lax
from jax.experimental import pallas as pl
from jax.experimental.pallas import tpu as pltpu
from jax.experimental.pallas import tpu_sc as plsc

_CH = 128


def _make_kernel(B, S, H, V):
    info = plsc.get_sparse_core_info()
    NC, NS = info.num_cores, info.num_subcores
    NW = NC * NS
    TOK = B * S
    TPW = TOK // NW
    CH = _CH
    NCH = TPW // CH
    G = H // 16

    mesh = plsc.VectorSubcoreMesh(core_axis_name="c", subcore_axis_name="s")

    @functools.partial(
        pl.kernel,
        out_type=jax.ShapeDtypeStruct((TOK, H), jnp.float32),
        mesh=mesh,
        compiler_params=pltpu.CompilerParams(use_tc_tiling_on_sc=False),
        scratch_types=[
            pltpu.VMEM((NCH, CH), jnp.int32),
            pltpu.VMEM((NCH, CH), jnp.int32),
            pltpu.VMEM((CH, H), jnp.float32),
            pltpu.VMEM((CH, H), jnp.float32),
            pltpu.VMEM((S, H), jnp.float32),
            pltpu.VMEM((2, H), jnp.float32),
            pltpu.SemaphoreType.DMA,
            pltpu.SemaphoreType.DMA,
            pltpu.SemaphoreType.DMA,
            pltpu.SemaphoreType.DMA,
        ],
    )
    def k(in_hbm, mask_hbm, emb_hbm, pos_hbm, mt_hbm, out_hbm,
          tall, mall, erow0, erow1, posv, mtv, semg0, semg1, semo0, semo1):
        wid = lax.axis_index("s") * NC + lax.axis_index("c")
        pltpu.sync_copy(pos_hbm, posv)
        pltpu.sync_copy(mt_hbm, mtv)
        pltpu.sync_copy(in_hbm.at[pl.ds(wid * NCH, NCH), :], tall)
        pltpu.sync_copy(mask_hbm.at[pl.ds(wid * NCH, NCH), :], mall)

        mt0 = [mtv[0, pl.ds(j * 16, 16)] for j in range(G)]
        d = [mtv[1, pl.ds(j * 16, 16)] - mt0[j] for j in range(G)]

        def pos_prep(s, carry):
            for j in range(G):
                sl = pl.ds(j * 16, 16)
                posv[s, sl] = posv[s, sl] + mt0[j]
            return carry

        lax.fori_loop(0, S, pos_prep, 0)

        erow = (erow0, erow1)
        semg = (semg0, semg1)
        semo = (semo0, semo1)

        def compute(c, p):
            def g_body(g, carry):
                mvec = mall[c, pl.ds(g * 16, 16)].astype(jnp.float32)
                for q in range(16):
                    t = g * 16 + q
                    pidx = lax.rem(c * CH + t, S)
                    mf = mvec[q]
                    for j in range(G):
                        sl = pl.ds(j * 16, 16)
                        erow[p][t, sl] = (erow[p][t, sl] + posv[pidx, sl]
                                          + mf * d[j])
                return carry

            lax.fori_loop(0, CH // 16, g_body, 0)

        def out_slice(c):
            return out_hbm.at[pl.ds(wid * TPW + c * CH, CH), :]

        def drain_gather(p):
            pltpu.make_async_copy(out_slice(0), erow[p], semg[p]).wait()

        def drain_out(c, p):
            pltpu.make_async_copy(erow[p], out_slice(c), semo[p]).wait()

        def stage(c, p):
            q = 1 - p

            @pl.when(c + 1 < NCH)
            def _():
                @pl.when(c >= 1)
                def _():
                    drain_out(c - 1, q)
                pltpu.async_copy(emb_hbm.at[tall.at[c + 1]], erow[q], semg[q])

            drain_gather(p)
            compute(c, p)
            pltpu.async_copy(erow[p], out_slice(c), semo[p])

        def pair_body(cc, carry):
            stage(2 * cc, 0)
            stage(2 * cc + 1, 1)
            return carry

        pltpu.async_copy(emb_hbm.at[tall.at[0]], erow0, semg0)
        lax.fori_loop(0, NCH // 2, pair_body, 0)
        drain_out(NCH - 2, 0)
        drain_out(NCH - 1, 1)

    return k


def kernel(Input, mask, emb_table, pos_table, mask_table):
    B, S = Input.shape
    V, H = emb_table.shape
    k = _make_kernel(B, S, H, V)
    out = k(Input.reshape(-1, _CH), mask.reshape(-1, _CH), emb_table,
            pos_table[:S], mask_table)
    return out.reshape(B, S, H)
